# baseline (speedup 1.0000x reference)
import functools

import jax
import jax.numpy as jnp
from jax import lax
from jax.experimental import pallas as pl
from jax.experimental.pallas import tpu as pltpu
from jax.experimental.pallas import tpu_sc as plsc

B = 16384
D = 64
NMETA = 32
H1 = 128
H2 = 64

_NC, _NS = 2, 16
_NW = _NC * _NS
_BPW = B // _NW


def _gather_body(uidx_hbm, iidx_hbm, utab_hbm, itab_hbm, uout_hbm, iout_hbm,
                 uidx_v, iidx_v, urows_v, irows_v, usem, isem):
    wid = lax.axis_index("s") * _NC + lax.axis_index("c")
    base = wid * _BPW
    pltpu.sync_copy(uidx_hbm.at[pl.ds(base, _BPW)], uidx_v)
    pltpu.sync_copy(iidx_hbm.at[pl.ds(base, _BPW)], iidx_v)
    ucp = pltpu.async_copy(utab_hbm.at[uidx_v], urows_v, usem)
    icp = pltpu.async_copy(itab_hbm.at[iidx_v], irows_v, isem)
    ucp.wait()
    icp.wait()
    pltpu.sync_copy(urows_v, uout_hbm.at[pl.ds(base, _BPW)])
    pltpu.sync_copy(irows_v, iout_hbm.at[pl.ds(base, _BPW)])


def _sc_gather(user_idx, item_idx, user_emb, item_emb):
    mesh = plsc.VectorSubcoreMesh(core_axis_name="c", subcore_axis_name="s")
    f = pl.kernel(
        _gather_body,
        mesh=mesh,
        out_type=[
            jax.ShapeDtypeStruct((B, D), jnp.float32),
            jax.ShapeDtypeStruct((B, D), jnp.float32),
        ],
        scratch_types=[
            pltpu.VMEM((_BPW,), jnp.int32),
            pltpu.VMEM((_BPW,), jnp.int32),
            pltpu.VMEM((_BPW, D), jnp.float32),
            pltpu.VMEM((_BPW, D), jnp.float32),
            pltpu.SemaphoreType.DMA,
            pltpu.SemaphoreType.DMA,
        ],
        compiler_params=pltpu.CompilerParams(use_tc_tiling_on_sc=False),
    )
    return f(user_idx, item_idx, user_emb, item_emb)


_BS = 2048


def _mlp_body(u_ref, i_ref, m_ref, w1u_ref, w1i_ref, w1m_ref, b1_ref,
              w2_ref, b2_ref, w3_ref, b3_ref, out_ref):
    h1 = jnp.dot(u_ref[...], w1u_ref[...], preferred_element_type=jnp.float32)
    h1 += jnp.dot(i_ref[...], w1i_ref[...], preferred_element_type=jnp.float32)
    h1 += jnp.dot(m_ref[...], w1m_ref[...], preferred_element_type=jnp.float32)
    h1 = jnp.maximum(h1 + b1_ref[...], 0.0)
    h2 = jnp.maximum(
        jnp.dot(h1, w2_ref[...], preferred_element_type=jnp.float32) + b2_ref[...],
        0.0)
    out = jnp.sum(h2 * w3_ref[...], axis=1) + b3_ref[0]
    out_ref[...] = out


def _tc_mlp(u, i, m, W1, b1, W2, b2, W3, b3):
    w1u = W1[:, :D].T
    w1i = W1[:, D:2 * D].T
    w1m = W1[:, 2 * D:].T
    b1r = b1.reshape(1, H1)
    w2t = W2.T
    b2r = b2.reshape(1, H2)
    w3r = W3.reshape(1, H2)
    full = lambda shape: pl.BlockSpec(shape, lambda b: (0,) * len(shape))
    return pl.pallas_call(
        _mlp_body,
        grid=(B // _BS,),
        in_specs=[
            pl.BlockSpec((_BS, D), lambda b: (b, 0)),
            pl.BlockSpec((_BS, D), lambda b: (b, 0)),
            pl.BlockSpec((_BS, NMETA), lambda b: (b, 0)),
            full((D, H1)),
            full((D, H1)),
            full((NMETA, H1)),
            full((1, H1)),
            full((H1, H2)),
            full((1, H2)),
            full((1, H2)),
            full((1,)),
        ],
        out_specs=pl.BlockSpec((_BS,), lambda b: (b,)),
        out_shape=jax.ShapeDtypeStruct((B,), jnp.float32),
    )(u, i, m, w1u, w1i, w1m, b1r, w2t, b2r, w3r, b3)


def kernel(user_idx, item_idx, metadata_vec, user_emb, item_emb,
           W1, b1, W2, b2, W3, b3):
    u, i = _sc_gather(user_idx, item_idx, user_emb, item_emb)
    return _tc_mlp(u, i, metadata_vec, W1, b1, W2, b2, W3, b3)


# speedup vs baseline: 1.0024x; 1.0024x over previous
import jax
import jax.numpy as jnp
from jax import lax
from jax.experimental import pallas as pl
from jax.experimental.pallas import tpu as pltpu
from jax.experimental.pallas import tpu_sc as plsc

B = 16384
D = 64
NMETA = 32
H1 = 128
H2 = 64

_NC, _NS = 2, 16
_NW = _NC * _NS
_BPW = B // _NW
_LANES = 16


def _gather_body(uidx_hbm, iidx_hbm, utab_hbm, itab_hbm, uout_hbm, iout_hbm,
                 uidx_v, iidx_v, rows_v, sem):
    wid = lax.axis_index("s") * _NC + lax.axis_index("c")
    base = wid * _BPW
    pltpu.sync_copy(uidx_hbm.at[pl.ds(base, _BPW)], uidx_v)
    pltpu.sync_copy(iidx_hbm.at[pl.ds(base, _BPW)], iidx_v)

    def halve(c, _):
        s = pl.ds(c * _LANES, _LANES)
        uidx_v[s] = jnp.right_shift(uidx_v[s], 1)
        iidx_v[s] = jnp.right_shift(iidx_v[s], 1)
        return ()

    lax.fori_loop(0, _BPW // _LANES, halve, (), unroll=4)

    pltpu.async_copy(utab_hbm.at[uidx_v], rows_v, sem).wait()
    pltpu.sync_copy(rows_v, uout_hbm.at[pl.ds(base, _BPW)])
    pltpu.async_copy(itab_hbm.at[iidx_v], rows_v, sem).wait()
    pltpu.sync_copy(rows_v, iout_hbm.at[pl.ds(base, _BPW)])


def _sc_gather(user_idx, item_idx, utab2, itab2):
    mesh = plsc.VectorSubcoreMesh(core_axis_name="c", subcore_axis_name="s")
    f = pl.kernel(
        _gather_body,
        mesh=mesh,
        out_type=[
            jax.ShapeDtypeStruct((B, 2 * D), jnp.float32),
            jax.ShapeDtypeStruct((B, 2 * D), jnp.float32),
        ],
        scratch_types=[
            pltpu.VMEM((_BPW,), jnp.int32),
            pltpu.VMEM((_BPW,), jnp.int32),
            pltpu.VMEM((_BPW, 2 * D), jnp.float32),
            pltpu.SemaphoreType.DMA,
        ],
    )
    return f(user_idx, item_idx, utab2, itab2)


_BS = 2048


def _mlp_body(u2_ref, i2_ref, up_ref, ip_ref, m_ref, w1u_ref, w1i_ref,
              w1m_ref, b1_ref, w2_ref, b2_ref, w3_ref, b3_ref, out_ref):
    up = up_ref[...]
    ip = ip_ref[...]
    ul = u2_ref[:, :D]
    ur = u2_ref[:, D:]
    il = i2_ref[:, :D]
    ir = i2_ref[:, D:]
    u = ul + (ur - ul) * up
    i = il + (ir - il) * ip
    h1 = jnp.dot(u, w1u_ref[...], preferred_element_type=jnp.float32)
    h1 += jnp.dot(i, w1i_ref[...], preferred_element_type=jnp.float32)
    h1 += jnp.dot(m_ref[...], w1m_ref[...], preferred_element_type=jnp.float32)
    h1 = jnp.maximum(h1 + b1_ref[...], 0.0)
    h2 = jnp.maximum(
        jnp.dot(h1, w2_ref[...], preferred_element_type=jnp.float32) + b2_ref[...],
        0.0)
    out = jnp.sum(h2 * w3_ref[...], axis=1) + b3_ref[0]
    out_ref[...] = out


def _tc_mlp(u2, i2, upar, ipar, m, W1, b1, W2, b2, W3, b3):
    w1u = W1[:, :D].T
    w1i = W1[:, D:2 * D].T
    w1m = W1[:, 2 * D:].T
    b1r = b1.reshape(1, H1)
    w2t = W2.T
    b2r = b2.reshape(1, H2)
    w3r = W3.reshape(1, H2)
    full = lambda shape: pl.BlockSpec(shape, lambda b: (0,) * len(shape))
    return pl.pallas_call(
        _mlp_body,
        grid=(B // _BS,),
        in_specs=[
            pl.BlockSpec((_BS, 2 * D), lambda b: (b, 0)),
            pl.BlockSpec((_BS, 2 * D), lambda b: (b, 0)),
            pl.BlockSpec((_BS, 1), lambda b: (b, 0)),
            pl.BlockSpec((_BS, 1), lambda b: (b, 0)),
            pl.BlockSpec((_BS, NMETA), lambda b: (b, 0)),
            full((D, H1)),
            full((D, H1)),
            full((NMETA, H1)),
            full((1, H1)),
            full((H1, H2)),
            full((1, H2)),
            full((1, H2)),
            full((1,)),
        ],
        out_specs=pl.BlockSpec((_BS,), lambda b: (b,)),
        out_shape=jax.ShapeDtypeStruct((B,), jnp.float32),
    )(u2, i2, upar, ipar, m, w1u, w1i, w1m, b1r, w2t, b2r, w3r, b3)


def kernel(user_idx, item_idx, metadata_vec, user_emb, item_emb,
           W1, b1, W2, b2, W3, b3):
    utab2 = user_emb.reshape(-1, 2 * D)
    itab2 = item_emb.reshape(-1, 2 * D)
    u2, i2 = _sc_gather(user_idx, item_idx, utab2, itab2)
    upar = (user_idx & 1).astype(jnp.float32).reshape(B, 1)
    ipar = (item_idx & 1).astype(jnp.float32).reshape(B, 1)
    return _tc_mlp(u2, i2, upar, ipar, metadata_vec, W1, b1, W2, b2, W3, b3)


# speedup vs baseline: 1.0042x; 1.0018x over previous
import jax
import jax.numpy as jnp
from jax import lax
from jax.experimental import pallas as pl
from jax.experimental.pallas import tpu as pltpu
from jax.experimental.pallas import tpu_sc as plsc

B = 16384
D = 64
NMETA = 32
H1 = 128
H2 = 64

_NC, _NS = 2, 16
_NW = _NC * _NS
_BPW = B // _NW
_LANES = 16


def _gather_body(uidx_hbm, iidx_hbm, utab_hbm, itab_hbm, uout_hbm, iout_hbm,
                 uidx_v, iidx_v, rows_v, sem):
    wid = lax.axis_index("s") * _NC + lax.axis_index("c")
    base = wid * _BPW
    pltpu.sync_copy(uidx_hbm.at[pl.ds(base, _BPW)], uidx_v)
    pltpu.sync_copy(iidx_hbm.at[pl.ds(base, _BPW)], iidx_v)

    def halve(c, _):
        s = pl.ds(c * _LANES, _LANES)
        uidx_v[s] = jnp.right_shift(uidx_v[s], 1)
        iidx_v[s] = jnp.right_shift(iidx_v[s], 1)
        return ()

    lax.fori_loop(0, _BPW // _LANES, halve, (), unroll=4)

    pltpu.async_copy(utab_hbm.at[uidx_v], rows_v, sem).wait()
    pltpu.sync_copy(rows_v, uout_hbm.at[pl.ds(base, _BPW)])
    pltpu.async_copy(itab_hbm.at[iidx_v], rows_v, sem).wait()
    pltpu.sync_copy(rows_v, iout_hbm.at[pl.ds(base, _BPW)])


def _sc_gather(user_idx, item_idx, utab2, itab2):
    mesh = plsc.VectorSubcoreMesh(core_axis_name="c", subcore_axis_name="s")
    f = pl.kernel(
        _gather_body,
        mesh=mesh,
        out_type=[
            jax.ShapeDtypeStruct((B, 2 * D), jnp.float32),
            jax.ShapeDtypeStruct((B, 2 * D), jnp.float32),
        ],
        scratch_types=[
            pltpu.VMEM((_BPW,), jnp.int32),
            pltpu.VMEM((_BPW,), jnp.int32),
            pltpu.VMEM((_BPW, 2 * D), jnp.float32),
            pltpu.SemaphoreType.DMA,
        ],
        compiler_params=pltpu.CompilerParams(use_tc_tiling_on_sc=True),
    )
    return f(user_idx, item_idx, utab2, itab2)


_BS = 2048


def _mlp_body(u2_ref, i2_ref, up_ref, ip_ref, m_ref, w1u_ref, w1i_ref,
              w1m_ref, b1_ref, w2_ref, b2_ref, w3_ref, b3_ref, out_ref):
    up = up_ref[...]
    ip = ip_ref[...]
    ul = u2_ref[:, :D]
    ur = u2_ref[:, D:]
    il = i2_ref[:, :D]
    ir = i2_ref[:, D:]
    u = ul + (ur - ul) * up
    i = il + (ir - il) * ip
    h1 = jnp.dot(u, w1u_ref[...], preferred_element_type=jnp.float32)
    h1 += jnp.dot(i, w1i_ref[...], preferred_element_type=jnp.float32)
    h1 += jnp.dot(m_ref[...], w1m_ref[...], preferred_element_type=jnp.float32)
    h1 = jnp.maximum(h1 + b1_ref[...], 0.0)
    h2 = jnp.maximum(
        jnp.dot(h1, w2_ref[...], preferred_element_type=jnp.float32) + b2_ref[...],
        0.0)
    out = jnp.sum(h2 * w3_ref[...], axis=1) + b3_ref[0]
    out_ref[...] = out


def _tc_mlp(u2, i2, upar, ipar, m, W1, b1, W2, b2, W3, b3):
    w1u = W1[:, :D].T
    w1i = W1[:, D:2 * D].T
    w1m = W1[:, 2 * D:].T
    b1r = b1.reshape(1, H1)
    w2t = W2.T
    b2r = b2.reshape(1, H2)
    w3r = W3.reshape(1, H2)
    full = lambda shape: pl.BlockSpec(shape, lambda b: (0,) * len(shape))
    return pl.pallas_call(
        _mlp_body,
        grid=(B // _BS,),
        in_specs=[
            pl.BlockSpec((_BS, 2 * D), lambda b: (b, 0)),
            pl.BlockSpec((_BS, 2 * D), lambda b: (b, 0)),
            pl.BlockSpec((_BS, 1), lambda b: (b, 0)),
            pl.BlockSpec((_BS, 1), lambda b: (b, 0)),
            pl.BlockSpec((_BS, NMETA), lambda b: (b, 0)),
            full((D, H1)),
            full((D, H1)),
            full((NMETA, H1)),
            full((1, H1)),
            full((H1, H2)),
            full((1, H2)),
            full((1, H2)),
            full((1,)),
        ],
        out_specs=pl.BlockSpec((_BS,), lambda b: (b,)),
        out_shape=jax.ShapeDtypeStruct((B,), jnp.float32),
    )(u2, i2, upar, ipar, m, w1u, w1i, w1m, b1r, w2t, b2r, w3r, b3)


def kernel(user_idx, item_idx, metadata_vec, user_emb, item_emb,
           W1, b1, W2, b2, W3, b3):
    utab2 = user_emb.reshape(-1, 2 * D)
    itab2 = item_emb.reshape(-1, 2 * D)
    u2, i2 = _sc_gather(user_idx, item_idx, utab2, itab2)
    upar = (user_idx & 1).astype(jnp.float32).reshape(B, 1)
    ipar = (item_idx & 1).astype(jnp.float32).reshape(B, 1)
    return _tc_mlp(u2, i2, upar, ipar, metadata_vec, W1, b1, W2, b2, W3, b3)


# speedup vs baseline: 2.6719x; 2.6607x over previous
import jax
import jax.numpy as jnp
from jax import lax
from jax.experimental import pallas as pl
from jax.experimental.pallas import tpu as pltpu
from jax.experimental.pallas import tpu_sc as plsc

B = 16384
D = 64
NMETA = 32
H1 = 128
H2 = 64
NROWS = 1000000

_NC, _NS = 2, 16
_NW = _NC * _NS
_BPW = B // _NW
_RING = 4
_CHUNK = 16
_NCHUNK = _BPW // _CHUNK
_FLUSH = 64


def _lane(vec, l):
    sel = jnp.where(lax.iota(jnp.int32, 16) == l, vec, 0)
    return jnp.sum(sel)


def _gather_body(uidx_hbm, iidx_hbm, utabT_hbm, itabT_hbm, x2_hbm,
                 uidx_v, iidx_v, stg_v,
                 ub0, ub1, ub2, ub3, ib0, ib1, ib2, ib3,
                 su0, su1, su2, su3, si0, si1, si2, si3):
    ubufs = (ub0, ub1, ub2, ub3)
    ibufs = (ib0, ib1, ib2, ib3)
    usems = (su0, su1, su2, su3)
    isems = (si0, si1, si2, si3)
    wid = lax.axis_index("s") * _NC + lax.axis_index("c")
    base = wid * _BPW
    pltpu.sync_copy(uidx_hbm.at[pl.ds(base, _BPW)], uidx_v)
    pltpu.sync_copy(iidx_hbm.at[pl.ds(base, _BPW)], iidx_v)

    def issue(tab, r, buf, sem):
        start = pl.multiple_of(lax.bitwise_and(r, jnp.int32(~127)), 128)
        pltpu.async_copy(tab.at[:, pl.ds(start, 128)], buf, sem)

    def wait(buf, sem):
        pltpu.make_async_copy(utabT_hbm.at[:, pl.ds(0, 128)], buf, sem).wait()

    def extract(buf, r, row, col0):
        lanev = jnp.broadcast_to(r - lax.bitwise_and(r, jnp.int32(~127)), (16,))
        for k in range(4):
            rows = lax.iota(jnp.int32, 16) + 16 * k
            vals = plsc.load_gather(buf, [rows, lanev])
            stg_v[row, pl.ds(col0 + 16 * k, 16)] = vals

    uvec0 = uidx_v[pl.ds(0, _CHUNK)]
    ivec0 = iidx_v[pl.ds(0, _CHUNK)]
    for s in range(_RING):
        issue(utabT_hbm, _lane(uvec0, s), ubufs[s], usems[s])
        issue(itabT_hbm, _lane(ivec0, s), ibufs[s], isems[s])

    def chunk(c, _):
        uvec = uidx_v[pl.ds(c * _CHUNK, _CHUNK)]
        ivec = iidx_v[pl.ds(c * _CHUNK, _CHUNK)]
        off_n = jnp.minimum((c + 1) * _CHUNK, _BPW - _CHUNK)
        uvec_n = uidx_v[pl.ds(off_n, _CHUNK)]
        ivec_n = iidx_v[pl.ds(off_n, _CHUNK)]
        row0 = 16 * lax.rem(c, 4)
        for l in range(_CHUNK):
            s = l % _RING
            ru = _lane(uvec, l)
            ri = _lane(ivec, l)
            wait(ubufs[s], usems[s])
            wait(ibufs[s], isems[s])
            extract(ubufs[s], ru, row0 + l, 0)
            extract(ibufs[s], ri, row0 + l, D)
            if l < _CHUNK - _RING:
                run = _lane(uvec, l + _RING)
                rin = _lane(ivec, l + _RING)
                issue(utabT_hbm, run, ubufs[s], usems[s])
                issue(itabT_hbm, rin, ibufs[s], isems[s])
            else:

                @pl.when(c < _NCHUNK - 1)
                def _():
                    run = _lane(uvec_n, l + _RING - _CHUNK)
                    rin = _lane(ivec_n, l + _RING - _CHUNK)
                    issue(utabT_hbm, run, ubufs[s], usems[s])
                    issue(itabT_hbm, rin, ibufs[s], isems[s])

        @pl.when(lax.rem(c, 4) == 3)
        def _():
            r0 = base + (c - 3) * _CHUNK
            pltpu.sync_copy(stg_v, x2_hbm.at[pl.ds(r0, _FLUSH)])

        return ()

    lax.fori_loop(0, _NCHUNK, chunk, ())


def _sc_gather(user_idx, item_idx, utabT, itabT):
    mesh = plsc.VectorSubcoreMesh(core_axis_name="c", subcore_axis_name="s")
    f = pl.kernel(
        _gather_body,
        mesh=mesh,
        out_type=jax.ShapeDtypeStruct((B, 2 * D), jnp.float32),
        scratch_types=[
            pltpu.VMEM((_BPW,), jnp.int32),
            pltpu.VMEM((_BPW,), jnp.int32),
            pltpu.VMEM((_FLUSH, 2 * D), jnp.float32),
        ] + [pltpu.VMEM((D, 128), jnp.float32)] * 8
          + [pltpu.SemaphoreType.DMA] * 8,
        compiler_params=pltpu.CompilerParams(use_tc_tiling_on_sc=True,
                                             needs_layout_passes=False),
    )
    return f(user_idx, item_idx, utabT, itabT)


_BS = 2048


def _mlp_body(x2_ref, m_ref, w1x_ref, w1m_ref, b1_ref, w2_ref, b2_ref,
              w3_ref, b3_ref, out_ref):
    h1 = jnp.dot(x2_ref[...], w1x_ref[...], preferred_element_type=jnp.float32)
    h1 += jnp.dot(m_ref[...], w1m_ref[...], preferred_element_type=jnp.float32)
    h1 = jnp.maximum(h1 + b1_ref[...], 0.0)
    h2 = jnp.maximum(
        jnp.dot(h1, w2_ref[...], preferred_element_type=jnp.float32)
        + b2_ref[...], 0.0)
    out_ref[...] = jnp.sum(h2 * w3_ref[...], axis=1) + b3_ref[0]


def _tc_mlp(x2, m, W1, b1, W2, b2, W3, b3):
    w1x = W1[:, :2 * D].T
    w1m = W1[:, 2 * D:].T
    b1r = b1.reshape(1, H1)
    w2t = W2.T
    b2r = b2.reshape(1, H2)
    w3r = W3.reshape(1, H2)
    full = lambda shape: pl.BlockSpec(shape, lambda b: (0,) * len(shape))
    return pl.pallas_call(
        _mlp_body,
        grid=(B // _BS,),
        in_specs=[
            pl.BlockSpec((_BS, 2 * D), lambda b: (b, 0)),
            pl.BlockSpec((_BS, NMETA), lambda b: (b, 0)),
            full((2 * D, H1)),
            full((NMETA, H1)),
            full((1, H1)),
            full((H1, H2)),
            full((1, H2)),
            full((1, H2)),
            full((1,)),
        ],
        out_specs=pl.BlockSpec((_BS,), lambda b: (b,)),
        out_shape=jax.ShapeDtypeStruct((B,), jnp.float32),
    )(x2, m, w1x, w1m, b1r, w2t, b2r, w3r, b3)


def kernel(user_idx, item_idx, metadata_vec, user_emb, item_emb,
           W1, b1, W2, b2, W3, b3):
    x2 = _sc_gather(user_idx, item_idx, user_emb.T, item_emb.T)
    return _tc_mlp(x2, metadata_vec, W1, b1, W2, b2, W3, b3)
